# baseline (speedup 1.0000x reference)
import functools

import jax
import jax.numpy as jnp
from jax import lax
from jax.experimental import pallas as pl
from jax.experimental.pallas import tpu as pltpu
from jax.experimental.pallas import tpu_sc as plsc

BATCH = 16384
EMBED = 64
POS = 10
NEG = 50
J = POS + NEG
JPAD = 64

_info = plsc.get_sparse_core_info()
_NC, _NS, _L = _info.num_cores, _info.num_subcores, _info.num_lanes
NW = _NC * _NS
BPW = BATCH // NW
C = 8
NCHUNK = BPW // C
STR_ROWS = 80
NPOS_STR = (C * POS) // STR_ROWS
NNEG_STR = (C * NEG) // STR_ROWS

TC_ROWS = 1024


def _sc_dots(in_rows, pos_labels, neg_labels, out_embed2):
    mesh = plsc.VectorSubcoreMesh(core_axis_name="c", subcore_axis_name="s")

    @functools.partial(
        pl.kernel,
        out_type=jax.ShapeDtypeStruct((BATCH * JPAD,), jnp.float32),
        mesh=mesh,
        compiler_params=pltpu.CompilerParams(
            needs_layout_passes=False, use_tc_tiling_on_sc=False),
        scratch_types=[
            pltpu.VMEM((2, C * POS + _L), jnp.int32),
            pltpu.VMEM((2, C * NEG + _L), jnp.int32),
            pltpu.VMEM((2, C * POS), jnp.int32),
            pltpu.VMEM((2, C * NEG), jnp.int32),
            pltpu.VMEM((2, C * EMBED), jnp.float32),
            pltpu.VMEM((2, C * POS, 2 * EMBED), jnp.float32),
            pltpu.VMEM((2, C * NEG, 2 * EMBED), jnp.float32),
            pltpu.VMEM((C * JPAD,), jnp.float32),
            pltpu.SemaphoreType.DMA,
            pltpu.SemaphoreType.DMA,
        ],
    )
    def k(in_rows_hbm, pos_lab_hbm, neg_lab_hbm, out_emb_hbm,
          dots_hbm, idx_pos_v, idx_neg_v, pidx_pos_v, pidx_neg_v,
          rows_in_v, rows_pos_v, rows_neg_v, dots_v, sem0, sem1):
        wid = lax.axis_index("s") * _NC + lax.axis_index("c")
        base = wid * BPW
        zero = jnp.zeros((_L,), jnp.float32)
        iota = lax.iota(jnp.int32, _L)
        onehot = [iota == l for l in range(_L)]
        sems = (sem0, sem1)

        def issue(g, k_):
            sem = sems[k_]
            b0 = base + g * C
            pltpu.sync_copy(pos_lab_hbm.at[pl.ds(b0 * POS, C * POS)],
                            idx_pos_v.at[k_, pl.ds(0, C * POS)])
            pltpu.sync_copy(neg_lab_hbm.at[pl.ds(b0 * NEG, C * NEG)],
                            idx_neg_v.at[k_, pl.ds(0, C * NEG)])
            for t in range(C * POS // _L):
                pidx_pos_v[k_, pl.ds(t * _L, _L)] = (
                    idx_pos_v[k_, pl.ds(t * _L, _L)] >> 1)
            for t in range(C * NEG // _L):
                pidx_neg_v[k_, pl.ds(t * _L, _L)] = (
                    idx_neg_v[k_, pl.ds(t * _L, _L)] >> 1)
            pltpu.async_copy(in_rows_hbm.at[pl.ds(b0 * EMBED, C * EMBED)],
                             rows_in_v.at[k_], sem)
            for s in range(NPOS_STR):
                pltpu.async_copy(
                    out_emb_hbm.at[pidx_pos_v.at[k_, pl.ds(s * STR_ROWS,
                                                           STR_ROWS)]],
                    rows_pos_v.at[k_, pl.ds(s * STR_ROWS, STR_ROWS)], sem)
            for s in range(NNEG_STR):
                pltpu.async_copy(
                    out_emb_hbm.at[pidx_neg_v.at[k_, pl.ds(s * STR_ROWS,
                                                           STR_ROWS)]],
                    rows_neg_v.at[k_, pl.ds(s * STR_ROWS, STR_ROWS)], sem)

        def drain(k_):
            sem = sems[k_]
            pltpu.make_async_copy(in_rows_hbm.at[pl.ds(0, C * EMBED)],
                                  rows_in_v.at[k_], sem).wait()
            pltpu.make_async_copy(out_emb_hbm.at[pl.ds(0, C * POS)],
                                  rows_pos_v.at[k_], sem).wait()
            pltpu.make_async_copy(out_emb_hbm.at[pl.ds(0, C * NEG)],
                                  rows_neg_v.at[k_], sem).wait()

        def compute(g, k_):
            def per_b(b, carry_b):
                ivecs = [rows_in_v[k_, pl.ds(b * EMBED + q * _L, _L)]
                         for q in range(4)]
                rp = b * POS
                rn = b * NEG
                accs = [zero, zero, zero, zero]
                for jj in range(J):
                    if jj < POS:
                        rv, iv_, r = rows_pos_v, idx_pos_v, rp + jj
                    else:
                        rv, iv_, r = rows_neg_v, idx_neg_v, rn + (jj - POS)
                    lab16 = iv_[k_, pl.ds(r, _L)]
                    off = (lab16[0] & 1) * EMBED
                    p = rv[k_, r, pl.ds(off, _L)] * ivecs[0]
                    for q in range(1, 4):
                        p = p + rv[k_, r, pl.ds(off + q * _L, _L)] * ivecs[q]
                    s = jnp.sum(p)
                    accs[jj // _L] = jnp.where(
                        onehot[jj % _L], jnp.broadcast_to(s, (_L,)),
                        accs[jj // _L])
                for q in range(4):
                    dots_v[pl.ds(b * JPAD + q * _L, _L)] = accs[q]
                return carry_b

            lax.fori_loop(0, C, per_b, 0)
            b0 = base + g * C
            pltpu.sync_copy(dots_v, dots_hbm.at[pl.ds(b0 * JPAD, C * JPAD)])

        issue(0, 0)

        def body(i, carry):
            g0 = 2 * i
            issue(g0 + 1, 1)
            drain(0)
            compute(g0, 0)
            issue(g0 + 2, 0)
            drain(1)
            compute(g0 + 1, 1)
            return carry

        lax.fori_loop(0, NCHUNK // 2 - 1, body, 0)
        issue(NCHUNK - 1, 1)
        drain(0)
        compute(NCHUNK - 2, 0)
        drain(1)
        compute(NCHUNK - 1, 1)

    return k(in_rows, pos_labels, neg_labels, out_embed2)


def _loss_body(dots_ref, out_ref):
    d = dots_ref[...].reshape(TC_ROWS, 128)
    lane = jax.lax.broadcasted_iota(jnp.int32, d.shape, 1)
    col = lane % JPAD
    sign = jnp.where(col < POS, 1.0, -1.0)
    x = d * sign
    ls = jnp.minimum(x, 0.0) - jnp.log1p(jnp.exp(-jnp.abs(x)))
    contrib = jnp.where(col < J, ls, 0.0)
    left = lane < JPAD
    s_even = jnp.sum(jnp.where(left, contrib, 0.0), axis=1, keepdims=True)
    s_odd = jnp.sum(jnp.where(left, 0.0, contrib), axis=1, keepdims=True)
    out_ref[...] = -jnp.concatenate([s_even, s_odd], axis=1)


def _loss_from_dots(dots_flat):
    nrows = BATCH * JPAD // 128
    out = pl.pallas_call(
        _loss_body,
        grid=(nrows // TC_ROWS,),
        in_specs=[pl.BlockSpec((TC_ROWS * 128,), lambda i: (i,))],
        out_specs=pl.BlockSpec((TC_ROWS, 2), lambda i: (i, 0)),
        out_shape=jax.ShapeDtypeStruct((nrows, 2), jnp.float32),
    )(dots_flat)
    return out.reshape(BATCH)


def kernel(input_labels, pos_labels, neg_labels, in_embed, out_embed):
    in_rows = jnp.take(in_embed, input_labels, axis=0).reshape(-1)
    dots = _sc_dots(in_rows,
                    pos_labels.reshape(-1).astype(jnp.int32),
                    neg_labels.reshape(-1).astype(jnp.int32),
                    out_embed.reshape(out_embed.shape[0] // 2, 2 * EMBED))
    return _loss_from_dots(dots)


# speedup vs baseline: 1.0712x; 1.0712x over previous
import functools

import jax
import jax.numpy as jnp
from jax import lax
from jax.experimental import pallas as pl
from jax.experimental.pallas import tpu as pltpu
from jax.experimental.pallas import tpu_sc as plsc

BATCH = 16384
EMBED = 64
POS = 10
NEG = 50
J = POS + NEG
JPAD = 64

_info = plsc.get_sparse_core_info()
_NC, _NS, _L = _info.num_cores, _info.num_subcores, _info.num_lanes
NW = _NC * _NS
BPW = BATCH // NW
C = 16
NCHUNK = BPW // C
STR_ROWS = 80
NPOS_STR = (C * POS) // STR_ROWS
NNEG_STR = (C * NEG) // STR_ROWS

TC_ROWS = 1024


def _sc_dots(in_rows, pos_labels, neg_labels, out_embed):
    mesh = plsc.VectorSubcoreMesh(core_axis_name="c", subcore_axis_name="s")

    @functools.partial(
        pl.kernel,
        out_type=jax.ShapeDtypeStruct((BATCH * JPAD,), jnp.float32),
        mesh=mesh,
        compiler_params=pltpu.CompilerParams(
            needs_layout_passes=False, use_tc_tiling_on_sc=False),
        scratch_types=[
            pltpu.VMEM((2, C * POS), jnp.int32),
            pltpu.VMEM((2, C * NEG), jnp.int32),
            pltpu.VMEM((2, C * EMBED), jnp.float32),
            pltpu.VMEM((2, C * POS, EMBED), jnp.float32),
            pltpu.VMEM((2, C * NEG, EMBED), jnp.float32),
            pltpu.VMEM((C * JPAD,), jnp.float32),
            pltpu.SemaphoreType.DMA,
            pltpu.SemaphoreType.DMA,
        ],
    )
    def k(in_rows_hbm, pos_lab_hbm, neg_lab_hbm, out_emb_hbm,
          dots_hbm, idx_pos_v, idx_neg_v, rows_in_v, rows_pos_v,
          rows_neg_v, dots_v, sem0, sem1):
        wid = lax.axis_index("s") * _NC + lax.axis_index("c")
        base = wid * BPW
        zero = jnp.zeros((_L,), jnp.float32)
        iota = lax.iota(jnp.int32, _L)
        onehot = [iota == l for l in range(_L)]
        sems = (sem0, sem1)

        def issue(g, k_):
            sem = sems[k_]
            b0 = base + g * C
            pltpu.sync_copy(pos_lab_hbm.at[pl.ds(b0 * POS, C * POS)],
                            idx_pos_v.at[k_])
            pltpu.sync_copy(neg_lab_hbm.at[pl.ds(b0 * NEG, C * NEG)],
                            idx_neg_v.at[k_])
            pltpu.async_copy(in_rows_hbm.at[pl.ds(b0 * EMBED, C * EMBED)],
                             rows_in_v.at[k_], sem)
            for s in range(NPOS_STR):
                pltpu.async_copy(
                    out_emb_hbm.at[idx_pos_v.at[k_, pl.ds(s * STR_ROWS,
                                                          STR_ROWS)]],
                    rows_pos_v.at[k_, pl.ds(s * STR_ROWS, STR_ROWS)], sem)
            for s in range(NNEG_STR):
                pltpu.async_copy(
                    out_emb_hbm.at[idx_neg_v.at[k_, pl.ds(s * STR_ROWS,
                                                          STR_ROWS)]],
                    rows_neg_v.at[k_, pl.ds(s * STR_ROWS, STR_ROWS)], sem)

        def drain(k_):
            sem = sems[k_]
            pltpu.make_async_copy(in_rows_hbm.at[pl.ds(0, C * EMBED)],
                                  rows_in_v.at[k_], sem).wait()
            pltpu.make_async_copy(out_emb_hbm.at[pl.ds(0, C * POS)],
                                  rows_pos_v.at[k_], sem).wait()
            pltpu.make_async_copy(out_emb_hbm.at[pl.ds(0, C * NEG)],
                                  rows_neg_v.at[k_], sem).wait()

        def compute(g, k_):
            def per_b(b, carry_b):
                ivecs = [rows_in_v[k_, pl.ds(b * EMBED + q * _L, _L)]
                         for q in range(4)]
                rp = b * POS
                rn = b * NEG
                accs = [zero, zero, zero, zero]
                for jj in range(J):
                    if jj < POS:
                        rv, r = rows_pos_v, rp + jj
                    else:
                        rv, r = rows_neg_v, rn + (jj - POS)
                    p = rv[k_, r, pl.ds(0, _L)] * ivecs[0]
                    for q in range(1, 4):
                        p = p + rv[k_, r, pl.ds(q * _L, _L)] * ivecs[q]
                    s = jnp.sum(p)
                    accs[jj // _L] = jnp.where(
                        onehot[jj % _L], jnp.broadcast_to(s, (_L,)),
                        accs[jj // _L])
                for q in range(4):
                    dots_v[pl.ds(b * JPAD + q * _L, _L)] = accs[q]
                return carry_b

            lax.fori_loop(0, C, per_b, 0)
            b0 = base + g * C
            pltpu.sync_copy(dots_v, dots_hbm.at[pl.ds(b0 * JPAD, C * JPAD)])

        issue(0, 0)

        def body(i, carry):
            g0 = 2 * i
            issue(g0 + 1, 1)
            drain(0)
            compute(g0, 0)
            issue(g0 + 2, 0)
            drain(1)
            compute(g0 + 1, 1)
            return carry

        lax.fori_loop(0, NCHUNK // 2 - 1, body, 0)
        issue(NCHUNK - 1, 1)
        drain(0)
        compute(NCHUNK - 2, 0)
        drain(1)
        compute(NCHUNK - 1, 1)

    return k(in_rows, pos_labels, neg_labels, out_embed)


def _loss_body(dots_ref, out_ref):
    d = dots_ref[...].reshape(TC_ROWS, 128)
    lane = jax.lax.broadcasted_iota(jnp.int32, d.shape, 1)
    col = lane % JPAD
    sign = jnp.where(col < POS, 1.0, -1.0)
    x = d * sign
    ls = jnp.minimum(x, 0.0) - jnp.log1p(jnp.exp(-jnp.abs(x)))
    contrib = jnp.where(col < J, ls, 0.0)
    left = lane < JPAD
    s_even = jnp.sum(jnp.where(left, contrib, 0.0), axis=1, keepdims=True)
    s_odd = jnp.sum(jnp.where(left, 0.0, contrib), axis=1, keepdims=True)
    out_ref[...] = -jnp.concatenate([s_even, s_odd], axis=1)


def _loss_from_dots(dots_flat):
    nrows = BATCH * JPAD // 128
    out = pl.pallas_call(
        _loss_body,
        grid=(nrows // TC_ROWS,),
        in_specs=[pl.BlockSpec((TC_ROWS * 128,), lambda i: (i,))],
        out_specs=pl.BlockSpec((TC_ROWS, 2), lambda i: (i, 0)),
        out_shape=jax.ShapeDtypeStruct((nrows, 2), jnp.float32),
    )(dots_flat)
    return out.reshape(BATCH)


def kernel(input_labels, pos_labels, neg_labels, in_embed, out_embed):
    in_rows = jnp.take(in_embed, input_labels, axis=0).reshape(-1)
    dots = _sc_dots(in_rows,
                    pos_labels.reshape(-1).astype(jnp.int32),
                    neg_labels.reshape(-1).astype(jnp.int32),
                    out_embed)
    return _loss_from_dots(dots)
